# Initial kernel scaffold; baseline (speedup 1.0000x reference)
import jax
import jax.numpy as jnp
from jax.experimental import pallas as pl


def kernel(x_t):
    raise NotImplementedError("write your pallas kernel here")


# speedup vs baseline: 8.2306x; 8.2306x over previous
import functools

import jax
import jax.numpy as jnp
from jax import lax
from jax.experimental import pallas as pl
from jax.experimental.pallas import tpu as pltpu
from jax.experimental.pallas import tpu_sc as plsc

DIM_ = 32
NLIM = DIM_ - 1


def _binidx_body(xa_ref, xb_ref, lim_ref, idx_ref):
    xa = xa_ref[0]
    xb = xb_ref[0]
    s = None
    for c in range(3):
        d = xa[:, c : c + 1] - xb[c : c + 1, :]
        t = d * d
        s = t if s is None else s + t
    dist = jnp.sqrt(s)
    idx = jnp.zeros(dist.shape, jnp.int32)
    for k in range(NLIM):
        idx += (dist > lim_ref[k]).astype(jnp.int32)
    idx_ref[0] = idx


def _bin_indices(x_t, limits):
    b, n, _ = x_t.shape
    pad = jnp.zeros((b, n, 5), x_t.dtype)
    xa = jnp.concatenate([x_t, pad], axis=-1)
    xb = jnp.transpose(xa, (0, 2, 1))
    return pl.pallas_call(
        _binidx_body,
        grid=(b,),
        in_specs=[
            pl.BlockSpec((1, n, 8), lambda i: (i, 0, 0)),
            pl.BlockSpec((1, 8, n), lambda i: (i, 0, 0)),
            pl.BlockSpec(memory_space=pltpu.SMEM),
        ],
        out_specs=pl.BlockSpec((1, n, n), lambda i: (i, 0, 0)),
        out_shape=jax.ShapeDtypeStruct((b, n, n), jnp.int32),
    )(xa, xb, limits)


NC = 2
NS = 16
NW = NC * NS
GCH = 128
NG = 8
CH = NG * GCH


def _make_expand(btot):
    b_per_w = btot // NW
    nchunk = b_per_w // CH
    mesh = plsc.VectorSubcoreMesh(
        core_axis_name="c", subcore_axis_name="s", num_cores=NC, num_subcores=NS
    )

    @functools.partial(
        pl.kernel,
        out_type=jax.ShapeDtypeStruct((btot, DIM_), jnp.float32),
        mesh=mesh,
        compiler_params=pltpu.CompilerParams(use_tc_tiling_on_sc=False),
        scratch_types=[
            pltpu.VMEM((NG, GCH), jnp.int32),
            pltpu.VMEM((CH, DIM_), jnp.float32),
            pltpu.SemaphoreType.DMA,
        ],
    )
    def expand(table_hbm, idx_hbm, out_hbm, idx_v, rows_v, sem):
        wid = lax.axis_index("s") * NC + lax.axis_index("c")
        base = wid * b_per_w

        def chunk(k, carry):
            off = pl.multiple_of(base + k * CH, CH)
            pltpu.sync_copy(idx_hbm.at[pl.ds(pl.multiple_of(off // GCH, NG), NG)], idx_v)
            copies = [
                pltpu.async_copy(
                    table_hbm.at[idx_v.at[g]],
                    rows_v.at[pl.ds(g * GCH, GCH)],
                    sem,
                )
                for g in range(NG)
            ]
            for c in copies:
                c.wait()
            pltpu.sync_copy(rows_v, out_hbm.at[pl.ds(off, CH)])
            return carry

        lax.fori_loop(0, nchunk, chunk, 0)

    return expand


def kernel(x_t):
    b, n, _ = x_t.shape
    limits = jnp.linspace(0.0, 20.0, NLIM)
    idx = _bin_indices(x_t, limits)
    btot = b * n * n
    idx2d = idx.reshape(btot // GCH, GCH)
    table = jnp.eye(DIM_, dtype=jnp.float32)
    out = _make_expand(btot)(table, idx2d)
    return out.reshape(b, n, n, DIM_)


# speedup vs baseline: 93.9819x; 11.4186x over previous
import functools

import jax
import jax.numpy as jnp
from jax import lax
from jax.experimental import pallas as pl
from jax.experimental.pallas import tpu as pltpu
from jax.experimental.pallas import tpu_sc as plsc

DIM_ = 32
NLIM = DIM_ - 1


def _binidx_body(xa_ref, xb_ref, lim_ref, idx_ref):
    xa = xa_ref[0]
    xb = xb_ref[0]
    s = None
    for c in range(3):
        d = xa[:, c : c + 1] - xb[c : c + 1, :]
        t = d * d
        s = t if s is None else s + t
    dist = jnp.sqrt(s)
    idx = jnp.zeros(dist.shape, jnp.int32)
    for k in range(NLIM):
        idx += (dist > lim_ref[k]).astype(jnp.int32)
    idx_ref[0] = idx


def _bin_indices(x_t, limits):
    b, n, _ = x_t.shape
    pad = jnp.zeros((b, n, 5), x_t.dtype)
    xa = jnp.concatenate([x_t, pad], axis=-1)
    xb = jnp.transpose(xa, (0, 2, 1))
    return pl.pallas_call(
        _binidx_body,
        grid=(b,),
        in_specs=[
            pl.BlockSpec((1, n, 8), lambda i: (i, 0, 0)),
            pl.BlockSpec((1, 8, n), lambda i: (i, 0, 0)),
            pl.BlockSpec(memory_space=pltpu.SMEM),
        ],
        out_specs=pl.BlockSpec((1, n, n), lambda i: (i, 0, 0)),
        out_shape=jax.ShapeDtypeStruct((b, n, n), jnp.int32),
    )(xa, xb, limits)


NC = 2
NS = 16
NW = NC * NS
CH = 1024
LANES = 16


def _make_expand(btot):
    b_per_w = btot // NW
    nchunk = b_per_w // CH
    chw = CH * DIM_
    mesh = plsc.VectorSubcoreMesh(
        core_axis_name="c", subcore_axis_name="s", num_cores=NC, num_subcores=NS
    )

    @functools.partial(
        pl.kernel,
        out_type=jax.ShapeDtypeStruct((btot * DIM_,), jnp.float32),
        mesh=mesh,
        compiler_params=pltpu.CompilerParams(
            use_tc_tiling_on_sc=False, needs_layout_passes=False
        ),
        scratch_types=[
            pltpu.VMEM((b_per_w,), jnp.int32),
            pltpu.VMEM((chw,), jnp.float32),
            pltpu.VMEM((chw,), jnp.float32),
            pltpu.SemaphoreType.DMA,
        ],
    )
    def expand(idx_hbm, out_hbm, idx_v, rows_a, rows_b, wsem):
        wid = lax.axis_index("s") * NC + lax.axis_index("c")
        base = pl.multiple_of(wid * b_per_w, b_per_w)
        pltpu.sync_copy(idx_hbm.at[pl.ds(base, b_per_w)], idx_v)

        zeros16 = jnp.zeros((LANES,), jnp.float32)
        ones16 = jnp.ones((LANES,), jnp.float32)
        lane_pos = lax.iota(jnp.int32, LANES) * DIM_

        bufs = (rows_a, rows_b)
        for buf in bufs:
            def zbody(i, c, buf=buf):
                buf[pl.ds(i * LANES, LANES)] = zeros16
                return c

            lax.fori_loop(0, chw // LANES, zbody, 0)

        obase = pl.multiple_of(base * DIM_, b_per_w * DIM_)
        for k in range(nchunk):
            buf = bufs[k % 2]
            if k >= 2:
                pltpu.make_async_copy(
                    buf, out_hbm.at[pl.ds(obase, chw)], wsem
                ).wait()
            km2 = max(k - 2, 0)

            def cbody(i, c, buf=buf, k=k, km2=km2):
                old = idx_v[pl.ds(km2 * CH + i * LANES, LANES)]
                new = idx_v[pl.ds(k * CH + i * LANES, LANES)]
                posb = lane_pos + i * (LANES * DIM_)
                plsc.store_scatter(buf, [posb + old], zeros16)
                plsc.store_scatter(buf, [posb + new], ones16)
                return c

            lax.fori_loop(0, CH // LANES, cbody, 0)
            pltpu.async_copy(
                buf,
                out_hbm.at[pl.ds(pl.multiple_of(obase + k * chw, chw), chw)],
                wsem,
            )
        for buf in bufs:
            pltpu.make_async_copy(
                buf, out_hbm.at[pl.ds(obase, chw)], wsem
            ).wait()

    return expand


def kernel(x_t):
    b, n, _ = x_t.shape
    limits = jnp.linspace(0.0, 20.0, NLIM)
    idx = _bin_indices(x_t, limits)
    btot = b * n * n
    out = _make_expand(btot)(idx.reshape(btot))
    return out.reshape(b, n, n, DIM_)


# speedup vs baseline: 526.0223x; 5.5971x over previous
import functools

import jax
import jax.numpy as jnp
from jax import lax
from jax.experimental import pallas as pl
from jax.experimental.pallas import tpu as pltpu
from jax.experimental.pallas import tpu_sc as plsc

DIM_ = 32
NLIM = DIM_ - 1


def _binidx_body(xa_ref, xb_ref, lim_ref, idx_ref):
    xa = xa_ref[0]
    xb = xb_ref[0]
    s = None
    for c in range(3):
        d = xa[:, c : c + 1] - xb[c : c + 1, :]
        t = d * d
        s = t if s is None else s + t
    dist = jnp.sqrt(s)
    idx = jnp.zeros(dist.shape, jnp.int32)
    for k in range(NLIM):
        idx += (dist > lim_ref[k]).astype(jnp.int32)
    idx_ref[0] = idx


def _bin_indices(x_t, limits):
    b, n, _ = x_t.shape
    pad = jnp.zeros((b, n, 5), x_t.dtype)
    xa = jnp.concatenate([x_t, pad], axis=-1)
    xb = jnp.transpose(xa, (0, 2, 1))
    return pl.pallas_call(
        _binidx_body,
        grid=(b,),
        in_specs=[
            pl.BlockSpec((1, n, 8), lambda i: (i, 0, 0)),
            pl.BlockSpec((1, 8, n), lambda i: (i, 0, 0)),
            pl.BlockSpec(memory_space=pltpu.SMEM),
        ],
        out_specs=pl.BlockSpec((1, n, n), lambda i: (i, 0, 0)),
        out_shape=jax.ShapeDtypeStruct((b, n, n), jnp.int32),
    )(xa, xb, limits)


NC = 2
NS = 16
NW = NC * NS
CH = 1024
LANES = 16


def _make_expand(btot, n):
    nface = btot // n
    fpc = CH // n
    b_per_w = btot // NW
    nchunk = b_per_w // CH
    f_per_w = nface // NW
    mesh = plsc.VectorSubcoreMesh(
        core_axis_name="c", subcore_axis_name="s", num_cores=NC, num_subcores=NS
    )

    @functools.partial(
        pl.kernel,
        out_type=jax.ShapeDtypeStruct((nface, DIM_, n), jnp.float32),
        mesh=mesh,
        compiler_params=pltpu.CompilerParams(
            use_tc_tiling_on_sc=True, needs_layout_passes=False
        ),
        scratch_types=[
            pltpu.VMEM((b_per_w,), jnp.int32),
            pltpu.VMEM((fpc, DIM_, n), jnp.float32),
            pltpu.VMEM((fpc, DIM_, n), jnp.float32),
            pltpu.SemaphoreType.DMA,
        ],
    )
    def expand(zeros_hbm, idx_hbm, out_hbm, idx_v, rows_a, rows_b, wsem):
        wid = lax.axis_index("s") * NC + lax.axis_index("c")
        base = pl.multiple_of(wid * b_per_w, b_per_w)
        pltpu.sync_copy(idx_hbm.at[pl.ds(base, b_per_w)], idx_v)

        zeros16 = jnp.zeros((LANES,), jnp.float32)
        ones16 = jnp.ones((LANES,), jnp.float32)
        lane = lax.iota(jnp.int32, LANES)
        gpf = n // LANES

        bufs = (rows_a, rows_b)
        for buf in bufs:
            pltpu.sync_copy(zeros_hbm, buf)

        fbase = pl.multiple_of(wid * f_per_w, f_per_w)
        for k in range(nchunk):
            buf = bufs[k % 2]
            if k >= 2:
                pltpu.make_async_copy(
                    buf, out_hbm.at[pl.ds(fbase, fpc)], wsem
                ).wait()
            km2 = max(k - 2, 0)

            def cbody(g, c, buf=buf, k=k, km2=km2):
                old = idx_v[pl.ds(km2 * CH + g * LANES, LANES)]
                new = idx_v[pl.ds(k * CH + g * LANES, LANES)]
                fvec = jnp.full((LANES,), g // gpf, jnp.int32)
                jvec = lane + (g % gpf) * LANES
                plsc.store_scatter(buf, [fvec, old, jvec], zeros16)
                plsc.store_scatter(buf, [fvec, new, jvec], ones16)
                return c

            lax.fori_loop(0, CH // LANES, cbody, 0)
            pltpu.async_copy(
                buf,
                out_hbm.at[pl.ds(pl.multiple_of(fbase + k * fpc, fpc), fpc)],
                wsem,
            )
        for buf in bufs:
            pltpu.make_async_copy(
                buf, out_hbm.at[pl.ds(fbase, fpc)], wsem
            ).wait()

    return expand


def kernel(x_t):
    b, n, _ = x_t.shape
    limits = jnp.linspace(0.0, 20.0, NLIM)
    idx = _bin_indices(x_t, limits)
    btot = b * n * n
    zblock = jnp.zeros((CH // n, DIM_, n), jnp.float32)
    out = _make_expand(btot, n)(zblock, idx.reshape(btot))
    return jnp.transpose(out.reshape(b, n, DIM_, n), (0, 1, 3, 2))


# speedup vs baseline: 635.3701x; 1.2079x over previous
import functools

import jax
import jax.numpy as jnp
from jax import lax
from jax.experimental import pallas as pl
from jax.experimental.pallas import tpu as pltpu
from jax.experimental.pallas import tpu_sc as plsc

DIM_ = 32
NLIM = DIM_ - 1


def _binidx_body(xa_ref, xb_ref, idx_ref):
    xa = xa_ref[0]
    xb = xb_ref[0]
    s = None
    for c in range(3):
        d = xa[:, c : c + 1] - xb[c : c + 1, :]
        t = d * d
        s = t if s is None else s + t
    dist = jnp.sqrt(s)
    idx = jnp.clip(jnp.ceil(dist * 1.5), 0.0, 31.0).astype(jnp.int32)
    idx_ref[0] = idx


def _bin_indices(x_t):
    b, n, _ = x_t.shape
    pad = jnp.zeros((b, n, 5), x_t.dtype)
    xa = jnp.concatenate([x_t, pad], axis=-1)
    xb = jnp.transpose(xa, (0, 2, 1))
    return pl.pallas_call(
        _binidx_body,
        grid=(b,),
        in_specs=[
            pl.BlockSpec((1, n, 8), lambda i: (i, 0, 0)),
            pl.BlockSpec((1, 8, n), lambda i: (i, 0, 0)),
        ],
        out_specs=pl.BlockSpec((1, n, n), lambda i: (i, 0, 0)),
        out_shape=jax.ShapeDtypeStruct((b, n, n), jnp.int32),
    )(xa, xb)


NC = 2
NS = 16
NW = NC * NS
CH = 1024
LANES = 16


def _make_expand(btot, n):
    nface = btot // n
    fpc = CH // n
    b_per_w = btot // NW
    nchunk = b_per_w // CH
    f_per_w = nface // NW
    mesh = plsc.VectorSubcoreMesh(
        core_axis_name="c", subcore_axis_name="s", num_cores=NC, num_subcores=NS
    )

    @functools.partial(
        pl.kernel,
        out_type=jax.ShapeDtypeStruct((nface, DIM_, n), jnp.float32),
        mesh=mesh,
        compiler_params=pltpu.CompilerParams(
            use_tc_tiling_on_sc=True, needs_layout_passes=False
        ),
        scratch_types=[
            pltpu.VMEM((f_per_w, n), jnp.int32),
            pltpu.VMEM((fpc, DIM_, n), jnp.float32),
            pltpu.VMEM((fpc, DIM_, n), jnp.float32),
            pltpu.SemaphoreType.DMA,
        ],
    )
    def expand(zeros_hbm, idx_hbm, out_hbm, idx_v, rows_a, rows_b, wsem):
        wid = lax.axis_index("s") * NC + lax.axis_index("c")
        pltpu.sync_copy(idx_hbm.at[wid], idx_v)

        zeros16 = jnp.zeros((LANES,), jnp.float32)
        ones16 = jnp.ones((LANES,), jnp.float32)
        lane = lax.iota(jnp.int32, LANES)
        gpf = n // LANES

        bufs = (rows_a, rows_b)
        for buf in bufs:
            pltpu.sync_copy(zeros_hbm, buf)

        fbase = pl.multiple_of(wid * f_per_w, f_per_w)
        for k in range(nchunk):
            buf = bufs[k % 2]
            if k >= 2:
                pltpu.make_async_copy(
                    buf, out_hbm.at[pl.ds(fbase, fpc)], wsem
                ).wait()
            km2 = max(k - 2, 0)

            def cbody(g, c, buf=buf, k=k, km2=km2):
                frel = g // gpf
                col = (g % gpf) * LANES
                old = idx_v[km2 * fpc + frel, pl.ds(col, LANES)]
                new = idx_v[k * fpc + frel, pl.ds(col, LANES)]
                fvec = jnp.full((LANES,), frel, jnp.int32)
                jvec = lane + col
                plsc.store_scatter(buf, [fvec, old, jvec], zeros16)
                plsc.store_scatter(buf, [fvec, new, jvec], ones16)
                return c

            lax.fori_loop(0, CH // LANES, cbody, 0)
            pltpu.async_copy(
                buf,
                out_hbm.at[pl.ds(pl.multiple_of(fbase + k * fpc, fpc), fpc)],
                wsem,
            )
        for buf in bufs:
            pltpu.make_async_copy(
                buf, out_hbm.at[pl.ds(fbase, fpc)], wsem
            ).wait()

    return expand


def kernel(x_t):
    b, n, _ = x_t.shape
    idx = _bin_indices(x_t)
    btot = b * n * n
    idx3 = idx.reshape(NW, btot // (NW * n), n)
    zblock = jnp.zeros((CH // n, DIM_, n), jnp.float32)
    out = _make_expand(btot, n)(zblock, idx3)
    return jnp.transpose(out.reshape(b, n, DIM_, n), (0, 1, 3, 2))
